# baseline (speedup 1.0000x reference)
import functools

import jax
import jax.numpy as jnp
from jax import lax
from jax.experimental import pallas as pl
from jax.experimental.pallas import tpu as pltpu
from jax.experimental.pallas import tpu_sc as plsc

N = 10000
E = 320000
D = 128
DE = 16

NC, NS = 2, 16
NW = NC * NS

_MESH = plsc.VectorSubcoreMesh(
    core_axis_name="c", subcore_axis_name="s", num_cores=NC, num_subcores=NS)


def _proj1_body(x_ref, w_ref, a_ref, b_ref):
    x = x_ref[...]
    ws = w_ref[128:256, :]
    wd = w_ref[0:128, :] - ws
    a_ref[...] = jnp.dot(x, wd, preferred_element_type=jnp.float32)
    b_ref[...] = jnp.dot(x, ws, preferred_element_type=jnp.float32)


def _proj1(x, w1):
    blk = 2000
    return pl.pallas_call(
        _proj1_body,
        grid=(N // blk,),
        in_specs=[pl.BlockSpec((blk, 128), lambda i: (i, 0)),
                  pl.BlockSpec((272, 128), lambda i: (0, 0))],
        out_specs=[pl.BlockSpec((blk, 128), lambda i: (i, 0))] * 2,
        out_shape=[jax.ShapeDtypeStruct((N, 128), jnp.float32)] * 2,
    )(x, w1)


def _proj2_body(p_ref, w_ref, a_ref, b_ref):
    x = jnp.maximum(p_ref[0], p_ref[1])
    ws = w_ref[128:256, :]
    wd = w_ref[0:128, :] - ws
    a_ref[...] = jnp.dot(x, wd, preferred_element_type=jnp.float32)
    b_ref[...] = jnp.dot(x, ws, preferred_element_type=jnp.float32)


def _proj2(p, w1):
    blk = 2000
    return pl.pallas_call(
        _proj2_body,
        grid=(N // blk,),
        in_specs=[pl.BlockSpec((2, blk, 128), lambda i: (0, i, 0)),
                  pl.BlockSpec((272, 128), lambda i: (0, 0))],
        out_specs=[pl.BlockSpec((blk, 128), lambda i: (i, 0))] * 2,
        out_shape=[jax.ShapeDtypeStruct((N, 128), jnp.float32)] * 2,
    )(p, w1)


def _mlp_body(g_ref, e_ref, w1e_ref, b1_ref, w2_ref, b2_ref, z_ref):
    h = (g_ref[...]
         + jnp.dot(e_ref[...], w1e_ref[...], preferred_element_type=jnp.float32)
         + b1_ref[...])
    h = jnp.maximum(h, 0.0)
    z = jnp.dot(h, w2_ref[...], preferred_element_type=jnp.float32) + b2_ref[...]
    z_ref[...] = jnp.maximum(z, 0.0)


def _mlp(g, eattr, w1e, b1, w2, b2):
    blk = 2560
    return pl.pallas_call(
        _mlp_body,
        grid=(E // blk,),
        in_specs=[pl.BlockSpec((blk, 128), lambda i: (i, 0)),
                  pl.BlockSpec((blk, 16), lambda i: (i, 0)),
                  pl.BlockSpec((16, 128), lambda i: (0, 0)),
                  pl.BlockSpec((1, 128), lambda i: (0, 0)),
                  pl.BlockSpec((128, 128), lambda i: (0, 0)),
                  pl.BlockSpec((1, 128), lambda i: (0, 0))],
        out_specs=pl.BlockSpec((blk, 128), lambda i: (i, 0)),
        out_shape=jax.ShapeDtypeStruct((E, 128), jnp.float32),
    )(g, eattr, w1e, b1, w2, b2)


def _head_body(p_ref, w3_ref, b3_ref, w4_ref, b4_ref, o_ref):
    x = jnp.maximum(p_ref[0], p_ref[1])
    h = jnp.dot(x, w3_ref[...], preferred_element_type=jnp.float32) + b3_ref[...]
    h = jnp.maximum(h, 0.0)
    o = jnp.dot(h, w4_ref[...], preferred_element_type=jnp.float32) + b4_ref[...]
    o_ref[...] = 1.0 / (1.0 + jnp.exp(-o))


def _head(p, w3, b3, w4p, b4p):
    blk = 2000
    return pl.pallas_call(
        _head_body,
        grid=(N // blk,),
        in_specs=[pl.BlockSpec((2, blk, 128), lambda i: (0, i, 0)),
                  pl.BlockSpec((128, 128), lambda i: (0, 0)),
                  pl.BlockSpec((1, 128), lambda i: (0, 0)),
                  pl.BlockSpec((128, 128), lambda i: (0, 0)),
                  pl.BlockSpec((1, 128), lambda i: (0, 0))],
        out_specs=pl.BlockSpec((blk, 128), lambda i: (i, 0)),
        out_shape=jax.ShapeDtypeStruct((N, 128), jnp.float32),
    )(p, w3, b3, w4p, b4p)


C_G = 200
EPW = E // NW
NCH_G = EPW // C_G


@functools.partial(
    pl.kernel,
    out_type=jax.ShapeDtypeStruct((E, 128), jnp.float32),
    mesh=_MESH,
    scratch_types=[
        pltpu.VMEM((C_G,), jnp.int32),
        pltpu.VMEM((C_G,), jnp.int32),
        pltpu.VMEM((C_G,), jnp.int32),
        pltpu.VMEM((C_G,), jnp.int32),
        pltpu.VMEM((C_G, 128), jnp.float32),
        pltpu.VMEM((C_G, 128), jnp.float32),
        pltpu.VMEM((C_G, 128), jnp.float32),
        pltpu.VMEM((C_G, 128), jnp.float32),
        pltpu.SemaphoreType.DMA,
        pltpu.SemaphoreType.DMA,
        pltpu.SemaphoreType.DMA,
        pltpu.SemaphoreType.DMA,
        pltpu.SemaphoreType.DMA,
        pltpu.SemaphoreType.DMA,
        pltpu.SemaphoreType.DMA,
        pltpu.SemaphoreType.DMA,
        pltpu.SemaphoreType.DMA,
        pltpu.SemaphoreType.DMA,
    ],
)
def _gather_k(a_hbm, b_hbm, dst_hbm, src_hbm, g_hbm,
              idxd0, idxd1, idxs0, idxs1, ra0, ra1, rb0, rb1,
              sga0, sga1, sgb0, sgb1, so0, so1, sid0, sid1, sis0, sis1):
    idxd = (idxd0, idxd1)
    idxs = (idxs0, idxs1)
    ra = (ra0, ra1)
    rb = (rb0, rb1)
    sga = (sga0, sga1)
    sgb = (sgb0, sgb1)
    so = (so0, so1)
    sid = (sid0, sid1)
    sis = (sis0, sis1)
    wid = lax.axis_index("s") * NC + lax.axis_index("c")
    base = wid * EPW

    def _wait_idx(p):
        pltpu.make_async_copy(dst_hbm.at[pl.ds(0, C_G)], idxd[p],
                              sid[p]).wait()
        pltpu.make_async_copy(src_hbm.at[pl.ds(0, C_G)], idxs[p],
                              sis[p]).wait()

    pltpu.sync_copy(dst_hbm.at[pl.ds(base, C_G)], idxd[0])
    pltpu.sync_copy(src_hbm.at[pl.ds(base, C_G)], idxs[0])
    pltpu.async_copy(a_hbm.at[idxd[0]], ra[0], sga[0])
    pltpu.async_copy(b_hbm.at[idxs[0]], rb[0], sgb[0])
    pltpu.async_copy(dst_hbm.at[pl.ds(base + C_G, C_G)], idxd[1], sid[1])
    pltpu.async_copy(src_hbm.at[pl.ds(base + C_G, C_G)], idxs[1], sis[1])

    @pl.loop(0, NCH_G // 2)
    def _cb(cb):
        for par in (0, 1):
            q = 1 - par
            cc = cb * 2 + par
            pltpu.make_async_copy(a_hbm.at[idxd[par]], ra[par],
                                  sga[par]).wait()
            pltpu.make_async_copy(b_hbm.at[idxs[par]], rb[par],
                                  sgb[par]).wait()
            if par == 1:
                pltpu.make_async_copy(ra[q], g_hbm.at[pl.ds(0, C_G), :],
                                      so[q]).wait()
            else:
                @pl.when(cc >= 1)
                def _w():
                    pltpu.make_async_copy(ra[q],
                                          g_hbm.at[pl.ds(0, C_G), :],
                                          so[q]).wait()
            _wait_idx(q)
            pltpu.async_copy(a_hbm.at[idxd[q]], ra[q], sga[q])
            pltpu.async_copy(b_hbm.at[idxs[q]], rb[q], sgb[q])

            @pl.loop(0, C_G)
            def _row(i):
                for j in range(8):
                    s = pl.ds(j * 16, 16)
                    ra[par][i, s] = ra[par][i, s] + rb[par][i, s]

            pltpu.async_copy(ra[par],
                             g_hbm.at[pl.ds(base + cc * C_G, C_G), :],
                             so[par])
            cc2 = jnp.minimum(cc + 2, NCH_G - 1)
            pltpu.async_copy(dst_hbm.at[pl.ds(base + cc2 * C_G, C_G)],
                             idxd[par], sid[par])
            pltpu.async_copy(src_hbm.at[pl.ds(base + cc2 * C_G, C_G)],
                             idxs[par], sis[par])

    pltpu.make_async_copy(a_hbm.at[idxd[0]], ra[0], sga[0]).wait()
    pltpu.make_async_copy(b_hbm.at[idxs[0]], rb[0], sgb[0]).wait()
    pltpu.make_async_copy(ra[1], g_hbm.at[pl.ds(0, C_G), :], so[1]).wait()
    _wait_idx(1)


C_S = 640
NG_S = C_S // 16
EHALF = E // 2
NCH_S = EHALF // C_S
NHALF = N // 2
NLIST = 4
NB = NHALF // NLIST
LCAP = C_S + 32
SENT = NB << 10


def _splat(vec, lane):
    idx = jnp.full((16, 1), lane, jnp.int32)
    dnums = lax.GatherDimensionNumbers(
        offset_dims=(), collapsed_slice_dims=(0,), start_index_map=(0,))
    return lax.gather(vec, idx, dnums, (1,),
                      mode=lax.GatherScatterMode.PROMISE_IN_BOUNDS)


@functools.partial(
    pl.kernel,
    out_type=jax.ShapeDtypeStruct((2, N, 128), jnp.float32),
    mesh=_MESH,
    scratch_types=[
        pltpu.VMEM((C_S,), jnp.int32),
        pltpu.VMEM((C_S,), jnp.int32),
        pltpu.VMEM((C_S, 16), jnp.float32),
        pltpu.VMEM((C_S, 16), jnp.float32),
    ] + [pltpu.VMEM((NB + 1, 16), jnp.float32)] * NLIST
      + [pltpu.VMEM((LCAP,), jnp.int32)] * NLIST
      + [
        pltpu.SemaphoreType.DMA,
        pltpu.SemaphoreType.DMA,
        pltpu.SemaphoreType.DMA,
        pltpu.SemaphoreType.DMA,
    ],
    compiler_params=pltpu.CompilerParams(use_tc_tiling_on_sc=False,
                                         needs_layout_passes=False),
)
def _scatter_k(z_hbm, dst_hbm, p_hbm, dstv0, dstv1, zv0, zv1,
               a0, a1, a2, a3, l0, l1, l2, l3, sd0, sd1, sz0, sz1):
    accs = (a0, a1, a2, a3)
    lists = (l0, l1, l2, l3)
    dstv = (dstv0, dstv1)
    zv = (zv0, zv1)
    sd = (sd0, sd1)
    sz = (sz0, sz1)
    wid = lax.axis_index("s") * NC + lax.axis_index("c")
    cg = wid % 8
    nh = (wid // 8) % 2
    eq = wid // 16
    lo = nh * NHALF
    iota = lax.iota(jnp.int32, 16)

    @pl.loop(0, NB + 1)
    def _zacc(i):
        z16 = jnp.zeros((16,), jnp.float32)
        for a in accs:
            a[i, :] = z16

    @pl.loop(0, LCAP // 16)
    def _zlst(i):
        z16i = jnp.zeros((16,), jnp.int32)
        for lst in lists:
            lst[pl.ds(i * 16, 16)] = z16i

    def _start_in(c, p):
        e0 = eq * EHALF + c * C_S
        pltpu.async_copy(dst_hbm.at[pl.ds(e0, C_S)], dstv[p], sd[p])
        pltpu.async_copy(z_hbm.at[pl.ds(e0, C_S), pl.ds(cg * 16, 16)],
                         zv[p], sz[p])

    def _wait_in(p):
        pltpu.make_async_copy(dst_hbm.at[pl.ds(0, C_S)], dstv[p],
                              sd[p]).wait()
        pltpu.make_async_copy(z_hbm.at[pl.ds(0, C_S), pl.ds(cg * 16, 16)],
                              zv[p], sz[p]).wait()

    def _process(par):
        dv = dstv[par]
        zvp = zv[par]

        @pl.loop(0, NG_S, init_carry=(jnp.int32(0),) * NLIST)
        def _split(j, cnts):
            dvec = dv[pl.ds(j * 16, 16)]
            rel = dvec - lo
            inr = plsc.bitcast(rel, jnp.uint32) < jnp.uint32(NHALF)
            ts = [rel >= q * NB for q in range(1, NLIST)]
            masks = ([inr & jnp.logical_not(ts[0])]
                     + [ts[q - 1] & jnp.logical_not(ts[q])
                        for q in range(1, NLIST - 1)]
                     + [ts[NLIST - 2] & inr])
            qnb = ts[0].astype(jnp.int32)
            for t in ts[1:]:
                qnb = qnb + t.astype(jnp.int32)
            packed = ((rel - qnb * NB) << 10) | (iota + j * 16)
            out = []
            for q in range(NLIST):
                plsc.store_compressed(lists[q].at[pl.ds(cnts[q], 16)],
                                      packed, mask=masks[q])
                pc = plsc.all_reduce_population_count(masks[q])
                out.append(cnts[q] + pc[0])
            return tuple(out)

        cnts = _split
        cmax = cnts[0]
        for cq in cnts[1:]:
            cmax = jnp.maximum(cmax, cq)
        ngrp = (cmax + 15) >> 4
        sent = jnp.full((16,), SENT, jnp.int32)
        for q in range(NLIST):
            @pl.loop(cnts[q], ngrp * 16, step=16)
            def _pad(i, q=q):
                lists[q][pl.ds(i, 16)] = sent

        @pl.loop(0, ngrp)
        def _rmw(k):
            base = k * 16
            pvs = [lst[pl.ds(base, 16)] for lst in lists]
            for l in range(16):
                for q in range(NLIST):
                    pk = _splat(pvs[q], l)
                    row = lax.shift_right_logical(pk, 10)
                    lidx = pk & 1023
                    zrow = plsc.load_gather(zvp, [lidx, iota])
                    old = plsc.load_gather(accs[q], [row, iota])
                    plsc.store_scatter(accs[q], [row, iota],
                                       jnp.maximum(old, zrow))

    _start_in(0, 0)

    @pl.loop(0, NCH_S // 2)
    def _chunk(cb):
        for par in (0, 1):
            q = 1 - par
            cc = cb * 2 + par
            _wait_in(par)
            _start_in(jnp.minimum(cc + 1, NCH_S - 1), q)
            _process(par)

    _wait_in(0)

    for q in range(NLIST):
        pltpu.sync_copy(accs[q].at[pl.ds(0, NB), :],
                        p_hbm.at[eq, pl.ds(lo + q * NB, NB),
                                 pl.ds(cg * 16, 16)])


def kernel(ajacency, node_features, edge_attributes,
           W1a, b1a, W2a, b2a, W1b, b1b, W2b, b2b, W3, b3, W4, b4):
    src = ajacency[0]
    dst = ajacency[1]

    b1a_r = b1a.reshape(1, 128)
    b2a_r = b2a.reshape(1, 128)
    b1b_r = b1b.reshape(1, 128)
    b2b_r = b2b.reshape(1, 128)
    b3_r = b3.reshape(1, 128)
    w1ea = W1a[256:272]
    w1eb = W1b[256:272]
    w4p = jnp.pad(W4, ((0, 0), (0, 127)))
    b4p = jnp.broadcast_to(b4.reshape(1, 1), (1, 128))

    a1, b1 = _proj1(node_features, W1a)
    g1 = _gather_k(a1, b1, dst, src)
    z1 = _mlp(g1, edge_attributes, w1ea, b1a_r, W2a, b2a_r)
    p1 = _scatter_k(z1, dst)

    a2, b2 = _proj2(p1, W1b)
    g2 = _gather_k(a2, b2, dst, src)
    z2 = _mlp(g2, edge_attributes, w1eb, b1b_r, W2b, b2b_r)
    p2 = _scatter_k(z2, dst)

    o = _head(p2, W3, b3_r, w4p, b4p)
    return o[:, 0:1]


# speedup vs baseline: 1.0141x; 1.0141x over previous
import functools

import jax
import jax.numpy as jnp
from jax import lax
from jax.experimental import pallas as pl
from jax.experimental.pallas import tpu as pltpu
from jax.experimental.pallas import tpu_sc as plsc

N = 10000
E = 320000
D = 128
DE = 16

NC, NS = 2, 16
NW = NC * NS

_MESH = plsc.VectorSubcoreMesh(
    core_axis_name="c", subcore_axis_name="s", num_cores=NC, num_subcores=NS)


def _proj1_body(x_ref, w_ref, a_ref, b_ref):
    x = x_ref[...]
    ws = w_ref[128:256, :]
    wd = w_ref[0:128, :] - ws
    a_ref[...] = jnp.dot(x, wd, preferred_element_type=jnp.float32)
    b_ref[...] = jnp.dot(x, ws, preferred_element_type=jnp.float32)


def _proj1(x, w1):
    blk = 2000
    return pl.pallas_call(
        _proj1_body,
        grid=(N // blk,),
        in_specs=[pl.BlockSpec((blk, 128), lambda i: (i, 0)),
                  pl.BlockSpec((272, 128), lambda i: (0, 0))],
        out_specs=[pl.BlockSpec((blk, 128), lambda i: (i, 0))] * 2,
        out_shape=[jax.ShapeDtypeStruct((N, 128), jnp.float32)] * 2,
    )(x, w1)


def _proj2_body(p_ref, w_ref, a_ref, b_ref):
    x = jnp.maximum(p_ref[0], p_ref[1])
    ws = w_ref[128:256, :]
    wd = w_ref[0:128, :] - ws
    a_ref[...] = jnp.dot(x, wd, preferred_element_type=jnp.float32)
    b_ref[...] = jnp.dot(x, ws, preferred_element_type=jnp.float32)


def _proj2(p, w1):
    blk = 2000
    return pl.pallas_call(
        _proj2_body,
        grid=(N // blk,),
        in_specs=[pl.BlockSpec((2, blk, 128), lambda i: (0, i, 0)),
                  pl.BlockSpec((272, 128), lambda i: (0, 0))],
        out_specs=[pl.BlockSpec((blk, 128), lambda i: (i, 0))] * 2,
        out_shape=[jax.ShapeDtypeStruct((N, 128), jnp.float32)] * 2,
    )(p, w1)


def _mlp_body(g_ref, e_ref, w1e_ref, b1_ref, w2_ref, b2_ref, z_ref):
    h = (g_ref[...]
         + jnp.dot(e_ref[...], w1e_ref[...], preferred_element_type=jnp.float32)
         + b1_ref[...])
    h = jnp.maximum(h, 0.0)
    z = jnp.dot(h, w2_ref[...], preferred_element_type=jnp.float32) + b2_ref[...]
    z_ref[...] = jnp.maximum(z, 0.0)


def _mlp(g, eattr, w1e, b1, w2, b2):
    blk = 2560
    return pl.pallas_call(
        _mlp_body,
        grid=(E // blk,),
        in_specs=[pl.BlockSpec((blk, 128), lambda i: (i, 0)),
                  pl.BlockSpec((blk, 16), lambda i: (i, 0)),
                  pl.BlockSpec((16, 128), lambda i: (0, 0)),
                  pl.BlockSpec((1, 128), lambda i: (0, 0)),
                  pl.BlockSpec((128, 128), lambda i: (0, 0)),
                  pl.BlockSpec((1, 128), lambda i: (0, 0))],
        out_specs=pl.BlockSpec((blk, 128), lambda i: (i, 0)),
        out_shape=jax.ShapeDtypeStruct((E, 128), jnp.float32),
    )(g, eattr, w1e, b1, w2, b2)


def _head_body(p_ref, w3_ref, b3_ref, w4_ref, b4_ref, o_ref):
    x = jnp.maximum(p_ref[0], p_ref[1])
    h = jnp.dot(x, w3_ref[...], preferred_element_type=jnp.float32) + b3_ref[...]
    h = jnp.maximum(h, 0.0)
    o = jnp.dot(h, w4_ref[...], preferred_element_type=jnp.float32) + b4_ref[...]
    o_ref[...] = 1.0 / (1.0 + jnp.exp(-o))


def _head(p, w3, b3, w4p, b4p):
    blk = 2000
    return pl.pallas_call(
        _head_body,
        grid=(N // blk,),
        in_specs=[pl.BlockSpec((2, blk, 128), lambda i: (0, i, 0)),
                  pl.BlockSpec((128, 128), lambda i: (0, 0)),
                  pl.BlockSpec((1, 128), lambda i: (0, 0)),
                  pl.BlockSpec((128, 128), lambda i: (0, 0)),
                  pl.BlockSpec((1, 128), lambda i: (0, 0))],
        out_specs=pl.BlockSpec((blk, 128), lambda i: (i, 0)),
        out_shape=jax.ShapeDtypeStruct((N, 128), jnp.float32),
    )(p, w3, b3, w4p, b4p)


C_G = 200
EPW = E // NW
NCH_G = EPW // C_G


@functools.partial(
    pl.kernel,
    out_type=jax.ShapeDtypeStruct((E, 128), jnp.float32),
    mesh=_MESH,
    scratch_types=[
        pltpu.VMEM((C_G,), jnp.int32),
        pltpu.VMEM((C_G,), jnp.int32),
        pltpu.VMEM((C_G,), jnp.int32),
        pltpu.VMEM((C_G,), jnp.int32),
        pltpu.VMEM((C_G, 128), jnp.float32),
        pltpu.VMEM((C_G, 128), jnp.float32),
        pltpu.VMEM((C_G, 128), jnp.float32),
        pltpu.VMEM((C_G, 128), jnp.float32),
        pltpu.SemaphoreType.DMA,
        pltpu.SemaphoreType.DMA,
        pltpu.SemaphoreType.DMA,
        pltpu.SemaphoreType.DMA,
        pltpu.SemaphoreType.DMA,
        pltpu.SemaphoreType.DMA,
        pltpu.SemaphoreType.DMA,
        pltpu.SemaphoreType.DMA,
        pltpu.SemaphoreType.DMA,
        pltpu.SemaphoreType.DMA,
    ],
)
def _gather_k(a_hbm, b_hbm, dst_hbm, src_hbm, g_hbm,
              idxd0, idxd1, idxs0, idxs1, ra0, ra1, rb0, rb1,
              sga0, sga1, sgb0, sgb1, so0, so1, sid0, sid1, sis0, sis1):
    idxd = (idxd0, idxd1)
    idxs = (idxs0, idxs1)
    ra = (ra0, ra1)
    rb = (rb0, rb1)
    sga = (sga0, sga1)
    sgb = (sgb0, sgb1)
    so = (so0, so1)
    sid = (sid0, sid1)
    sis = (sis0, sis1)
    wid = lax.axis_index("s") * NC + lax.axis_index("c")
    base = wid * EPW

    def _wait_idx(p):
        pltpu.make_async_copy(dst_hbm.at[pl.ds(0, C_G)], idxd[p],
                              sid[p]).wait()
        pltpu.make_async_copy(src_hbm.at[pl.ds(0, C_G)], idxs[p],
                              sis[p]).wait()

    pltpu.sync_copy(dst_hbm.at[pl.ds(base, C_G)], idxd[0])
    pltpu.sync_copy(src_hbm.at[pl.ds(base, C_G)], idxs[0])
    pltpu.async_copy(a_hbm.at[idxd[0]], ra[0], sga[0])
    pltpu.async_copy(b_hbm.at[idxs[0]], rb[0], sgb[0])
    pltpu.async_copy(dst_hbm.at[pl.ds(base + C_G, C_G)], idxd[1], sid[1])
    pltpu.async_copy(src_hbm.at[pl.ds(base + C_G, C_G)], idxs[1], sis[1])

    @pl.loop(0, NCH_G // 2)
    def _cb(cb):
        for par in (0, 1):
            q = 1 - par
            cc = cb * 2 + par
            pltpu.make_async_copy(a_hbm.at[idxd[par]], ra[par],
                                  sga[par]).wait()
            pltpu.make_async_copy(b_hbm.at[idxs[par]], rb[par],
                                  sgb[par]).wait()
            if par == 1:
                pltpu.make_async_copy(ra[q], g_hbm.at[pl.ds(0, C_G), :],
                                      so[q]).wait()
            else:
                @pl.when(cc >= 1)
                def _w():
                    pltpu.make_async_copy(ra[q],
                                          g_hbm.at[pl.ds(0, C_G), :],
                                          so[q]).wait()
            _wait_idx(q)
            pltpu.async_copy(a_hbm.at[idxd[q]], ra[q], sga[q])
            pltpu.async_copy(b_hbm.at[idxs[q]], rb[q], sgb[q])

            @pl.loop(0, C_G)
            def _row(i):
                for j in range(8):
                    s = pl.ds(j * 16, 16)
                    ra[par][i, s] = ra[par][i, s] + rb[par][i, s]

            pltpu.async_copy(ra[par],
                             g_hbm.at[pl.ds(base + cc * C_G, C_G), :],
                             so[par])
            cc2 = jnp.minimum(cc + 2, NCH_G - 1)
            pltpu.async_copy(dst_hbm.at[pl.ds(base + cc2 * C_G, C_G)],
                             idxd[par], sid[par])
            pltpu.async_copy(src_hbm.at[pl.ds(base + cc2 * C_G, C_G)],
                             idxs[par], sis[par])

    pltpu.make_async_copy(a_hbm.at[idxd[0]], ra[0], sga[0]).wait()
    pltpu.make_async_copy(b_hbm.at[idxs[0]], rb[0], sgb[0]).wait()
    pltpu.make_async_copy(ra[1], g_hbm.at[pl.ds(0, C_G), :], so[1]).wait()
    _wait_idx(1)


C_S = 800
NG_S = C_S // 16
EHALF = E // 2
NCH_S = EHALF // C_S
NHALF = N // 2
NLIST = 4
NB = NHALF // NLIST
LCAP = C_S + 32
SENT = NB << 10


def _splat(vec, lane):
    idx = jnp.full((16, 1), lane, jnp.int32)
    dnums = lax.GatherDimensionNumbers(
        offset_dims=(), collapsed_slice_dims=(0,), start_index_map=(0,))
    return lax.gather(vec, idx, dnums, (1,),
                      mode=lax.GatherScatterMode.PROMISE_IN_BOUNDS)


@functools.partial(
    pl.kernel,
    out_type=jax.ShapeDtypeStruct((2, N, 128), jnp.float32),
    mesh=_MESH,
    scratch_types=[
        pltpu.VMEM((C_S,), jnp.int32),
        pltpu.VMEM((C_S,), jnp.int32),
        pltpu.VMEM((C_S, 16), jnp.float32),
        pltpu.VMEM((C_S, 16), jnp.float32),
    ] + [pltpu.VMEM((NB + 1, 16), jnp.float32)] * NLIST
      + [pltpu.VMEM((LCAP,), jnp.int32)] * NLIST
      + [
        pltpu.SemaphoreType.DMA,
        pltpu.SemaphoreType.DMA,
        pltpu.SemaphoreType.DMA,
        pltpu.SemaphoreType.DMA,
    ],
    compiler_params=pltpu.CompilerParams(use_tc_tiling_on_sc=False,
                                         needs_layout_passes=False),
)
def _scatter_k(z_hbm, dst_hbm, p_hbm, dstv0, dstv1, zv0, zv1,
               a0, a1, a2, a3, l0, l1, l2, l3, sd0, sd1, sz0, sz1):
    accs = (a0, a1, a2, a3)
    lists = (l0, l1, l2, l3)
    dstv = (dstv0, dstv1)
    zv = (zv0, zv1)
    sd = (sd0, sd1)
    sz = (sz0, sz1)
    wid = lax.axis_index("s") * NC + lax.axis_index("c")
    cg = wid % 8
    nh = (wid // 8) % 2
    eq = wid // 16
    lo = nh * NHALF
    iota = lax.iota(jnp.int32, 16)

    @pl.loop(0, NB + 1)
    def _zacc(i):
        z16 = jnp.zeros((16,), jnp.float32)
        for a in accs:
            a[i, :] = z16

    @pl.loop(0, LCAP // 16)
    def _zlst(i):
        z16i = jnp.zeros((16,), jnp.int32)
        for lst in lists:
            lst[pl.ds(i * 16, 16)] = z16i

    def _start_in(c, p):
        e0 = eq * EHALF + c * C_S
        pltpu.async_copy(dst_hbm.at[pl.ds(e0, C_S)], dstv[p], sd[p])
        pltpu.async_copy(z_hbm.at[pl.ds(e0, C_S), pl.ds(cg * 16, 16)],
                         zv[p], sz[p])

    def _wait_in(p):
        pltpu.make_async_copy(dst_hbm.at[pl.ds(0, C_S)], dstv[p],
                              sd[p]).wait()
        pltpu.make_async_copy(z_hbm.at[pl.ds(0, C_S), pl.ds(cg * 16, 16)],
                              zv[p], sz[p]).wait()

    def _process(par):
        dv = dstv[par]
        zvp = zv[par]

        @pl.loop(0, NG_S, init_carry=(jnp.int32(0),) * NLIST)
        def _split(j, cnts):
            dvec = dv[pl.ds(j * 16, 16)]
            rel = dvec - lo
            inr = plsc.bitcast(rel, jnp.uint32) < jnp.uint32(NHALF)
            ts = [rel >= q * NB for q in range(1, NLIST)]
            masks = ([inr & jnp.logical_not(ts[0])]
                     + [ts[q - 1] & jnp.logical_not(ts[q])
                        for q in range(1, NLIST - 1)]
                     + [ts[NLIST - 2] & inr])
            qnb = ts[0].astype(jnp.int32)
            for t in ts[1:]:
                qnb = qnb + t.astype(jnp.int32)
            packed = ((rel - qnb * NB) << 10) | (iota + j * 16)
            out = []
            for q in range(NLIST):
                plsc.store_compressed(lists[q].at[pl.ds(cnts[q], 16)],
                                      packed, mask=masks[q])
                pc = plsc.all_reduce_population_count(masks[q])
                out.append(cnts[q] + pc[0])
            return tuple(out)

        cnts = _split
        cmax = cnts[0]
        for cq in cnts[1:]:
            cmax = jnp.maximum(cmax, cq)
        ngrp = (cmax + 15) >> 4
        sent = jnp.full((16,), SENT, jnp.int32)
        for q in range(NLIST):
            @pl.loop(cnts[q], ngrp * 16, step=16)
            def _pad(i, q=q):
                lists[q][pl.ds(i, 16)] = sent

        @pl.loop(0, ngrp)
        def _rmw(k):
            base = k * 16
            pvs = [lst[pl.ds(base, 16)] for lst in lists]
            for l in range(16):
                for q in range(NLIST):
                    pk = _splat(pvs[q], l)
                    row = lax.shift_right_logical(pk, 10)
                    lidx = pk & 1023
                    zrow = plsc.load_gather(zvp, [lidx, iota])
                    old = plsc.load_gather(accs[q], [row, iota])
                    plsc.store_scatter(accs[q], [row, iota],
                                       jnp.maximum(old, zrow))

    _start_in(0, 0)

    @pl.loop(0, NCH_S // 2)
    def _chunk(cb):
        for par in (0, 1):
            q = 1 - par
            cc = cb * 2 + par
            _wait_in(par)
            _start_in(jnp.minimum(cc + 1, NCH_S - 1), q)
            _process(par)

    _wait_in(0)

    for q in range(NLIST):
        pltpu.sync_copy(accs[q].at[pl.ds(0, NB), :],
                        p_hbm.at[eq, pl.ds(lo + q * NB, NB),
                                 pl.ds(cg * 16, 16)])


def kernel(ajacency, node_features, edge_attributes,
           W1a, b1a, W2a, b2a, W1b, b1b, W2b, b2b, W3, b3, W4, b4):
    src = ajacency[0]
    dst = ajacency[1]

    b1a_r = b1a.reshape(1, 128)
    b2a_r = b2a.reshape(1, 128)
    b1b_r = b1b.reshape(1, 128)
    b2b_r = b2b.reshape(1, 128)
    b3_r = b3.reshape(1, 128)
    w1ea = W1a[256:272]
    w1eb = W1b[256:272]
    w4p = jnp.pad(W4, ((0, 0), (0, 127)))
    b4p = jnp.broadcast_to(b4.reshape(1, 1), (1, 128))

    a1, b1 = _proj1(node_features, W1a)
    g1 = _gather_k(a1, b1, dst, src)
    z1 = _mlp(g1, edge_attributes, w1ea, b1a_r, W2a, b2a_r)
    p1 = _scatter_k(z1, dst)

    a2, b2 = _proj2(p1, W1b)
    g2 = _gather_k(a2, b2, dst, src)
    z2 = _mlp(g2, edge_attributes, w1eb, b1b_r, W2b, b2b_r)
    p2 = _scatter_k(z2, dst)

    o = _head(p2, W3, b3_r, w4p, b4p)
    return o[:, 0:1]
